# Initial kernel scaffold; baseline (speedup 1.0000x reference)
import jax
import jax.numpy as jnp
from jax.experimental import pallas as pl


def kernel(x, edge_index, adj_vals, W0, b0, a0, W1, b1, a1, Wm, bm, Ws, bs):
    raise NotImplementedError("write your pallas kernel here")


# speedup vs baseline: 5.6310x; 5.6310x over previous
import jax
import jax.numpy as jnp
from jax import lax
from jax.experimental import pallas as pl
from jax.experimental.pallas import tpu as pltpu
from jax.experimental.pallas import tpu_sc as plsc

N = 10000
NPAD = 10240
E = 160000
D = 128
NZ = 16

NC = 2
NS = 16
NT = NC * NS
NQ = 4
NF = NZ // NQ
NE = NT // NQ
EP = 163840
EPW = EP // NE
BLK = 5120
NB = EPW // BLK

BM = 200


def _encode0_body(x_ref, w_ref, b_ref, o_ref):
    h = (jnp.dot(x_ref[...], w_ref[...], preferred_element_type=jnp.float32)
         + b_ref[...])
    o_ref[:, :N] = h.T


def _combine(p_ref):
    quarters = []
    for q in range(NQ):
        wids = [c * NS + s for c in range(NC) for s in range(NS) if s % NQ == q]
        sq = p_ref[wids[0], :, :]
        for w in wids[1:]:
            sq = sq + p_ref[w, :, :]
        quarters.append(sq)
    return jnp.concatenate(quarters, axis=0)


def _mid_body(p_ref, wt_ref, b_ref, a_ref, o_ref):
    s = _combine(p_ref)
    h = jnp.where(s >= 0, s, s * a_ref[...])
    o_ref[...] = (
        jnp.dot(wt_ref[...], h, preferred_element_type=jnp.float32) + b_ref[...]
    )


def _heads_body(p_ref, wmt_ref, bm_ref, wst_ref, bs_ref, a_ref, noise_t_ref,
                zm_ref, zs_ref, z_ref, zt_ref):
    s = _combine(p_ref)
    h = jnp.where(s >= 0, s, s * a_ref[...])
    zmt = (jnp.dot(wmt_ref[...], h, preferred_element_type=jnp.float32)
           + bm_ref[...])
    v = (jnp.dot(wst_ref[...], h, preferred_element_type=jnp.float32)
         + bs_ref[...])
    zst = jnp.maximum(v, 0.0) + jnp.log(1.0 + jnp.exp(-jnp.abs(v)))
    zmt = zmt[:, :N]
    zst = zst[:, :N]
    zt = zmt + noise_t_ref[...] * zst
    zm_ref[...] = zmt.T
    zs_ref[...] = zst.T
    z_ref[...] = zt.T
    zt_ref[...] = zt


def _decode_body(z_ref, zt_ref, o_ref):
    logits = jnp.dot(z_ref[...], zt_ref[...], preferred_element_type=jnp.float32)
    o_ref[...] = 1.0 / (1.0 + jnp.exp(-logits))


_encode0 = pl.pallas_call(
    _encode0_body,
    out_shape=jax.ShapeDtypeStruct((NZ, NPAD), jnp.float32),
)

_mid = pl.pallas_call(
    _mid_body,
    out_shape=jax.ShapeDtypeStruct((NZ, NPAD), jnp.float32),
)

_heads = pl.pallas_call(
    _heads_body,
    out_shape=(
        jax.ShapeDtypeStruct((N, NZ), jnp.float32),
        jax.ShapeDtypeStruct((N, NZ), jnp.float32),
        jax.ShapeDtypeStruct((N, NZ), jnp.float32),
        jax.ShapeDtypeStruct((NZ, N), jnp.float32),
    ),
)

_decode = pl.pallas_call(
    _decode_body,
    grid=(N // BM,),
    in_specs=[
        pl.BlockSpec((BM, NZ), lambda i: (i, 0)),
        pl.BlockSpec((NZ, N), lambda i: (0, 0)),
    ],
    out_specs=pl.BlockSpec((BM, N), lambda i: (i, 0)),
    out_shape=jax.ShapeDtypeStruct((N, N), jnp.float32),
)


def _spmm_body(ht_hbm, src_hbm, dst_hbm, adj_hbm, out_hbm,
               src_v, dst_v, adj_v, h_v, acc_v):
    cid = lax.axis_index("c")
    sid = lax.axis_index("s")
    q = sid % NQ
    es = cid * NQ + sid // NQ
    wid = cid * NS + sid

    pltpu.sync_copy(ht_hbm.at[pl.ds(q * (NF * NPAD), NF * NPAD)], h_v)

    zero16 = jnp.zeros((16,), jnp.float32)

    def _zb(i, c):
        acc_v[pl.ds(i * 16, 16)] = zero16
        return c
    lax.fori_loop(0, (NF * NPAD) // 16, _zb, 0)

    lanesf = [jnp.full((16,), f * NPAD, jnp.int32) for f in range(NF)]

    def _block(b, c):
        off = es * EPW + b * BLK
        pltpu.sync_copy(src_hbm.at[pl.ds(off, BLK)], src_v)
        pltpu.sync_copy(dst_hbm.at[pl.ds(off, BLK)], dst_v)
        pltpu.sync_copy(adj_hbm.at[pl.ds(off, BLK)], adj_v)

        def _grp(g, cc):
            src16 = src_v[pl.ds(g * 16, 16)]
            dst16 = dst_v[pl.ds(g * 16, 16)]
            adj16 = adj_v[pl.ds(g * 16, 16)]
            for f in range(NF):
                vals = plsc.load_gather(h_v, [src16 + lanesf[f]]) * adj16
                plsc.addupdate_scatter(acc_v, [dst16 + lanesf[f]], vals)
            return cc
        lax.fori_loop(0, BLK // 16, _grp, 0)
        return c
    lax.fori_loop(0, NB, _block, 0)

    for f in range(NF):
        pltpu.sync_copy(acc_v.at[pl.ds(f * NPAD, NPAD)], out_hbm.at[wid, f])


_SPMM_CACHE = {}


def _get_spmm():
    if "k" not in _SPMM_CACHE:
        _SPMM_CACHE["k"] = pl.kernel(
            _spmm_body,
            mesh=plsc.VectorSubcoreMesh(
                core_axis_name="c", subcore_axis_name="s", num_cores=NC),
            compiler_params=pltpu.CompilerParams(needs_layout_passes=False),
            out_type=jax.ShapeDtypeStruct((NT, NF, NPAD), jnp.float32),
            scratch_types=[
                pltpu.VMEM((BLK,), jnp.int32),
                pltpu.VMEM((BLK,), jnp.int32),
                pltpu.VMEM((BLK,), jnp.float32),
                pltpu.VMEM((NF * NPAD,), jnp.float32),
                pltpu.VMEM((NF * NPAD,), jnp.float32),
            ],
        )
    return _SPMM_CACHE["k"]


def kernel(x, edge_index, adj_vals, W0, b0, a0, W1, b1, a1, Wm, bm, Ws, bs):
    dst = edge_index[0]
    src = edge_index[1]
    pad = EP - E
    srcp = jnp.pad(src, (0, pad))
    dstp = jnp.pad(dst, (0, pad))
    adjp = jnp.pad(adj_vals, (0, pad))
    noise_t = jax.random.normal(jax.random.key(42), (N, NZ),
                                dtype=jnp.float32).T

    spmm = _get_spmm()

    def run_spmm(ht):
        return spmm(ht.reshape(-1), srcp, dstp, adjp)

    h0t = _encode0(x, W0, b0.reshape(1, NZ))
    parts0 = run_spmm(h0t)
    h1t = _mid(parts0, W1.T, b1.reshape(NZ, 1), a0.reshape(1, 1))
    parts1 = run_spmm(h1t)
    zm, zs, z, zt = _heads(parts1, Wm.T, bm.reshape(NZ, 1), Ws.T,
                           bs.reshape(NZ, 1), a1.reshape(1, 1), noise_t)
    a_probs = _decode(z, zt)
    return (a_probs, zm, zs)


# speedup vs baseline: 6.3290x; 1.1240x over previous
import jax
import jax.numpy as jnp
from jax import lax
from jax.experimental import pallas as pl
from jax.experimental.pallas import tpu as pltpu
from jax.experimental.pallas import tpu_sc as plsc

N = 10000
NPAD = 10240
E = 160000
D = 128
NZ = 16

NC = 2
NS = 16
NT = NC * NS
NQ = 4
NF = NZ // NQ
NE = NT // NQ
EP = 163840
EPW = EP // NE
BLK = 5120
NB = EPW // BLK

BM = 200


def _encode0_body(x_ref, w_ref, b_ref, o_ref):
    h = (jnp.dot(x_ref[...], w_ref[...], preferred_element_type=jnp.float32)
         + b_ref[...])
    o_ref[:, :N] = h.T


def _combine(p_ref):
    quarters = []
    for q in range(NQ):
        wids = [c * NS + s for c in range(NC) for s in range(NS) if s % NQ == q]
        sq = p_ref[wids[0], :, :]
        for w in wids[1:]:
            sq = sq + p_ref[w, :, :]
        quarters.append(sq)
    return jnp.concatenate(quarters, axis=0)


def _mid_body(p_ref, wt_ref, b_ref, a_ref, o_ref):
    s = _combine(p_ref)
    h = jnp.where(s >= 0, s, s * a_ref[...])
    o_ref[...] = (
        jnp.dot(wt_ref[...], h, preferred_element_type=jnp.float32) + b_ref[...]
    )


def _heads_body(p_ref, wmt_ref, bm_ref, wst_ref, bs_ref, a_ref, noise_t_ref,
                zm_ref, zs_ref, z_ref, zt_ref):
    s = _combine(p_ref)
    h = jnp.where(s >= 0, s, s * a_ref[...])
    zmt = (jnp.dot(wmt_ref[...], h, preferred_element_type=jnp.float32)
           + bm_ref[...])
    v = (jnp.dot(wst_ref[...], h, preferred_element_type=jnp.float32)
         + bs_ref[...])
    zst = jnp.maximum(v, 0.0) + jnp.log(1.0 + jnp.exp(-jnp.abs(v)))
    zmt = zmt[:, :N]
    zst = zst[:, :N]
    zt = zmt + noise_t_ref[...] * zst
    zm_ref[...] = zmt.T
    zs_ref[...] = zst.T
    z_ref[...] = zt.T
    zt_ref[...] = zt


def _decode_body(z_ref, zt_ref, o_ref):
    logits = jnp.dot(z_ref[...], zt_ref[...], preferred_element_type=jnp.float32)
    o_ref[...] = 1.0 / (1.0 + jnp.exp(-logits))


_encode0 = pl.pallas_call(
    _encode0_body,
    out_shape=jax.ShapeDtypeStruct((NZ, NPAD), jnp.float32),
)

_mid = pl.pallas_call(
    _mid_body,
    out_shape=jax.ShapeDtypeStruct((NZ, NPAD), jnp.float32),
)

_heads = pl.pallas_call(
    _heads_body,
    out_shape=(
        jax.ShapeDtypeStruct((N, NZ), jnp.float32),
        jax.ShapeDtypeStruct((N, NZ), jnp.float32),
        jax.ShapeDtypeStruct((N, NZ), jnp.float32),
        jax.ShapeDtypeStruct((NZ, N), jnp.float32),
    ),
)

_decode = pl.pallas_call(
    _decode_body,
    grid=(N // BM,),
    in_specs=[
        pl.BlockSpec((BM, NZ), lambda i: (i, 0)),
        pl.BlockSpec((NZ, N), lambda i: (0, 0)),
    ],
    out_specs=pl.BlockSpec((BM, N), lambda i: (i, 0)),
    out_shape=jax.ShapeDtypeStruct((N, N), jnp.float32),
)


UNROLL = 4


def _spmm_body(ht_hbm, src_hbm, dst_hbm, adj_hbm, out_hbm,
               src_a, dst_a, adj_a, src_b, dst_b, adj_b, h_v, acc_v,
               sem_h, sem_e0, sem_e1):
    cid = lax.axis_index("c")
    sid = lax.axis_index("s")
    q = sid % NQ
    es = cid * NQ + sid // NQ
    wid = cid * NS + sid

    hcopy = pltpu.async_copy(
        ht_hbm.at[pl.ds(q * (NF * NPAD), NF * NPAD)], h_v, sem_h)

    bufs = [(src_a, dst_a, adj_a), (src_b, dst_b, adj_b)]
    sems = [sem_e0, sem_e1]

    def _start(b):
        off = es * EPW + b * BLK
        sv, dv, av = bufs[b % 2]
        sem = sems[b % 2]
        return (pltpu.async_copy(src_hbm.at[pl.ds(off, BLK)], sv, sem),
                pltpu.async_copy(dst_hbm.at[pl.ds(off, BLK)], dv, sem),
                pltpu.async_copy(adj_hbm.at[pl.ds(off, BLK)], av, sem))

    pending = {0: _start(0)}

    zero16 = jnp.zeros((16,), jnp.float32)

    def _zb(i, c):
        for u in range(8):
            acc_v[pl.ds((i * 8 + u) * 16, 16)] = zero16
        return c
    lax.fori_loop(0, (NF * NPAD) // (16 * 8), _zb, 0)
    hcopy.wait()

    lanesf = [jnp.full((16,), f * NPAD, jnp.int32) for f in range(NF)]

    for b in range(NB):
        if b + 1 < NB:
            pending[b + 1] = _start(b + 1)
        for c in pending.pop(b):
            c.wait()
        sv, dv, av = bufs[b % 2]

        def _grp(g, cc, sv=sv, dv=dv, av=av):
            for u in range(UNROLL):
                base = (g * UNROLL + u) * 16
                src16 = sv[pl.ds(base, 16)]
                dst16 = dv[pl.ds(base, 16)]
                adj16 = av[pl.ds(base, 16)]
                for f in range(NF):
                    vals = plsc.load_gather(h_v, [src16 + lanesf[f]]) * adj16
                    plsc.addupdate_scatter(acc_v, [dst16 + lanesf[f]], vals)
            return cc
        lax.fori_loop(0, BLK // (16 * UNROLL), _grp, 0)

    for f in range(NF):
        pltpu.sync_copy(acc_v.at[pl.ds(f * NPAD, NPAD)], out_hbm.at[wid, f])


_SPMM_CACHE = {}


def _get_spmm():
    if "k" not in _SPMM_CACHE:
        _SPMM_CACHE["k"] = pl.kernel(
            _spmm_body,
            mesh=plsc.VectorSubcoreMesh(
                core_axis_name="c", subcore_axis_name="s", num_cores=NC),
            compiler_params=pltpu.CompilerParams(needs_layout_passes=False),
            out_type=jax.ShapeDtypeStruct((NT, NF, NPAD), jnp.float32),
            scratch_types=[
                pltpu.VMEM((BLK,), jnp.int32),
                pltpu.VMEM((BLK,), jnp.int32),
                pltpu.VMEM((BLK,), jnp.float32),
                pltpu.VMEM((BLK,), jnp.int32),
                pltpu.VMEM((BLK,), jnp.int32),
                pltpu.VMEM((BLK,), jnp.float32),
                pltpu.VMEM((NF * NPAD,), jnp.float32),
                pltpu.VMEM((NF * NPAD,), jnp.float32),
                pltpu.SemaphoreType.DMA,
                pltpu.SemaphoreType.DMA,
                pltpu.SemaphoreType.DMA,
            ],
        )
    return _SPMM_CACHE["k"]


def kernel(x, edge_index, adj_vals, W0, b0, a0, W1, b1, a1, Wm, bm, Ws, bs):
    dst = edge_index[0]
    src = edge_index[1]
    pad = EP - E
    srcp = jnp.pad(src, (0, pad))
    dstp = jnp.pad(dst, (0, pad))
    adjp = jnp.pad(adj_vals, (0, pad))
    noise_t = jax.random.normal(jax.random.key(42), (N, NZ),
                                dtype=jnp.float32).T

    spmm = _get_spmm()

    def run_spmm(ht):
        return spmm(ht.reshape(-1), srcp, dstp, adjp)

    h0t = _encode0(x, W0, b0.reshape(1, NZ))
    parts0 = run_spmm(h0t)
    h1t = _mid(parts0, W1.T, b1.reshape(NZ, 1), a0.reshape(1, 1))
    parts1 = run_spmm(h1t)
    zm, zs, z, zt = _heads(parts1, Wm.T, bm.reshape(NZ, 1), Ws.T,
                           bs.reshape(NZ, 1), a1.reshape(1, 1), noise_t)
    a_probs = _decode(z, zt)
    return (a_probs, zm, zs)


# speedup vs baseline: 7.8464x; 1.2398x over previous
import jax
import jax.numpy as jnp
from jax import lax
from jax.experimental import pallas as pl
from jax.experimental.pallas import tpu as pltpu
from jax.experimental.pallas import tpu_sc as plsc

N = 10000
NPAD = 10240
E = 160000
D = 128
NZ = 16

NC = 2
NS = 16
NT = NC * NS
NQ = 4
NF = NZ // NQ
NE = NT // NQ
EP = 163840
EPW = EP // NE
BLK = 5120
NB = EPW // BLK

BM = 400


def _encode0_body(x_ref, w_ref, b_ref, o_ref):
    h = (jnp.dot(x_ref[...], w_ref[...], preferred_element_type=jnp.float32)
         + b_ref[...])
    o_ref[:, :N] = h.T


def _combine(p_ref):
    quarters = []
    for q in range(NQ):
        wids = [c * NS + s for c in range(NC) for s in range(NS) if s % NQ == q]
        sq = p_ref[wids[0], :, :]
        for w in wids[1:]:
            sq = sq + p_ref[w, :, :]
        quarters.append(sq)
    return jnp.concatenate(quarters, axis=0)


def _mid_body(p_ref, wt_ref, b_ref, a_ref, o_ref):
    s = _combine(p_ref)
    h = jnp.where(s >= 0, s, s * a_ref[...])
    o_ref[...] = (
        jnp.dot(wt_ref[...], h, preferred_element_type=jnp.float32) + b_ref[...]
    )


def _heads_body(p_ref, wmt_ref, bm_ref, wst_ref, bs_ref, a_ref, noise_t_ref,
                zm_ref, zs_ref, z_ref, zt_ref):
    s = _combine(p_ref)
    h = jnp.where(s >= 0, s, s * a_ref[...])
    zmt = (jnp.dot(wmt_ref[...], h, preferred_element_type=jnp.float32)
           + bm_ref[...])
    v = (jnp.dot(wst_ref[...], h, preferred_element_type=jnp.float32)
         + bs_ref[...])
    zst = jnp.maximum(v, 0.0) + jnp.log(1.0 + jnp.exp(-jnp.abs(v)))
    zmt = zmt[:, :N]
    zst = zst[:, :N]
    zt = zmt + noise_t_ref[...] * zst
    zm_ref[...] = zmt.T
    zs_ref[...] = zst.T
    z_ref[...] = zt.T
    zt_ref[...] = zt


def _decode_body(z_ref, zt_ref, o_ref):
    logits = jnp.dot(z_ref[...], zt_ref[...], preferred_element_type=jnp.float32)
    o_ref[...] = 1.0 / (1.0 + jnp.exp(-logits))


_encode0 = pl.pallas_call(
    _encode0_body,
    out_shape=jax.ShapeDtypeStruct((NZ, NPAD), jnp.float32),
)

_mid = pl.pallas_call(
    _mid_body,
    out_shape=jax.ShapeDtypeStruct((NZ, NPAD), jnp.float32),
)

_heads = pl.pallas_call(
    _heads_body,
    out_shape=(
        jax.ShapeDtypeStruct((N, NZ), jnp.float32),
        jax.ShapeDtypeStruct((N, NZ), jnp.float32),
        jax.ShapeDtypeStruct((N, NZ), jnp.float32),
        jax.ShapeDtypeStruct((NZ, N), jnp.float32),
    ),
)

_decode = pl.pallas_call(
    _decode_body,
    grid=(N // BM,),
    in_specs=[
        pl.BlockSpec((BM, NZ), lambda i: (i, 0)),
        pl.BlockSpec((NZ, N), lambda i: (0, 0)),
    ],
    out_specs=pl.BlockSpec((BM, N), lambda i: (i, 0)),
    out_shape=jax.ShapeDtypeStruct((N, N), jnp.float32),
)


UNROLL = 4


def _spmm_body(ht_hbm, src_hbm, dst_hbm, adj_hbm, out_hbm,
               src_a, dst_a, adj_a, src_b, dst_b, adj_b, h_v, acc_v,
               sem_h, sem_e0, sem_e1):
    cid = lax.axis_index("c")
    sid = lax.axis_index("s")
    q = sid % NQ
    es = cid * NQ + sid // NQ
    wid = cid * NS + sid

    hcopy = pltpu.async_copy(
        ht_hbm.at[pl.ds(q * (NF * NPAD), NF * NPAD)], h_v, sem_h)

    bufs = [(src_a, dst_a, adj_a), (src_b, dst_b, adj_b)]
    sems = [sem_e0, sem_e1]

    def _start(b):
        off = es * EPW + b * BLK
        sv, dv, av = bufs[b % 2]
        sem = sems[b % 2]
        return (pltpu.async_copy(src_hbm.at[pl.ds(off, BLK)], sv, sem),
                pltpu.async_copy(dst_hbm.at[pl.ds(off, BLK)], dv, sem),
                pltpu.async_copy(adj_hbm.at[pl.ds(off, BLK)], av, sem))

    pending = {0: _start(0)}

    zero16 = jnp.zeros((16,), jnp.float32)

    def _zb(i, c):
        for u in range(8):
            acc_v[pl.ds((i * 8 + u) * 16, 16)] = zero16
        return c
    lax.fori_loop(0, (NF * NPAD) // (16 * 8), _zb, 0)
    hcopy.wait()

    lanesf = [jnp.full((16,), f * NPAD, jnp.int32) for f in range(NF)]

    for b in range(NB):
        if b + 1 < NB:
            pending[b + 1] = _start(b + 1)
        for c in pending.pop(b):
            c.wait()
        sv, dv, av = bufs[b % 2]

        @plsc.parallel_loop(0, BLK // 16, unroll=UNROLL)
        def _grp(g, sv=sv, dv=dv, av=av):
            base = g * 16
            src16 = sv[pl.ds(base, 16)]
            dst16 = dv[pl.ds(base, 16)]
            adj16 = av[pl.ds(base, 16)]
            for f in range(NF):
                vals = plsc.load_gather(h_v, [src16 + lanesf[f]]) * adj16
                plsc.addupdate_scatter(acc_v, [dst16 + lanesf[f]], vals)

    for f in range(NF):
        pltpu.sync_copy(acc_v.at[pl.ds(f * NPAD, NPAD)], out_hbm.at[wid, f])


_SPMM_CACHE = {}


def _get_spmm():
    if "k" not in _SPMM_CACHE:
        _SPMM_CACHE["k"] = pl.kernel(
            _spmm_body,
            mesh=plsc.VectorSubcoreMesh(
                core_axis_name="c", subcore_axis_name="s", num_cores=NC),
            compiler_params=pltpu.CompilerParams(needs_layout_passes=False),
            out_type=jax.ShapeDtypeStruct((NT, NF, NPAD), jnp.float32),
            scratch_types=[
                pltpu.VMEM((BLK,), jnp.int32),
                pltpu.VMEM((BLK,), jnp.int32),
                pltpu.VMEM((BLK,), jnp.float32),
                pltpu.VMEM((BLK,), jnp.int32),
                pltpu.VMEM((BLK,), jnp.int32),
                pltpu.VMEM((BLK,), jnp.float32),
                pltpu.VMEM((NF * NPAD,), jnp.float32),
                pltpu.VMEM((NF * NPAD,), jnp.float32),
                pltpu.SemaphoreType.DMA,
                pltpu.SemaphoreType.DMA,
                pltpu.SemaphoreType.DMA,
            ],
        )
    return _SPMM_CACHE["k"]


def kernel(x, edge_index, adj_vals, W0, b0, a0, W1, b1, a1, Wm, bm, Ws, bs):
    dst = edge_index[0]
    src = edge_index[1]
    pad = EP - E
    srcp = jnp.pad(src, (0, pad))
    dstp = jnp.pad(dst, (0, pad))
    adjp = jnp.pad(adj_vals, (0, pad))
    noise_t = jax.random.normal(jax.random.key(42), (N, NZ),
                                dtype=jnp.float32).T

    spmm = _get_spmm()

    def run_spmm(ht):
        return spmm(ht.reshape(-1), srcp, dstp, adjp)

    h0t = _encode0(x, W0, b0.reshape(1, NZ))
    parts0 = run_spmm(h0t)
    h1t = _mid(parts0, W1.T, b1.reshape(NZ, 1), a0.reshape(1, 1))
    parts1 = run_spmm(h1t)
    zm, zs, z, zt = _heads(parts1, Wm.T, bm.reshape(NZ, 1), Ws.T,
                           bs.reshape(NZ, 1), a1.reshape(1, 1), noise_t)
    a_probs = _decode(z, zt)
    return (a_probs, zm, zs)


# speedup vs baseline: 8.6746x; 1.1056x over previous
import jax
import jax.numpy as jnp
from jax import lax
from jax.experimental import pallas as pl
from jax.experimental.pallas import tpu as pltpu
from jax.experimental.pallas import tpu_sc as plsc

N = 10000
NPAD = 10240
E = 160000
D = 128
NZ = 16

NC = 2
NS = 16
NT = NC * NS
NQ = 4
NF = NZ // NQ
NE = NT // NQ
EP = 163840
EPW = EP // NE
BLK = 5120
NB = EPW // BLK

BM = 400


def _encode0_body(x_ref, w_ref, b_ref, o_ref):
    h = (jnp.dot(x_ref[...], w_ref[...], preferred_element_type=jnp.float32)
         + b_ref[...])
    o_ref[:, :N] = h.T


def _combine(p_ref):
    quarters = []
    for q in range(NQ):
        wids = [c * NS + s for c in range(NC) for s in range(NS) if s % NQ == q]
        sq = p_ref[wids[0], :, :]
        for w in wids[1:]:
            sq = sq + p_ref[w, :, :]
        quarters.append(sq)
    return jnp.concatenate(quarters, axis=0)


def _mid_body(p_ref, wt_ref, b_ref, a_ref, o_ref):
    s = _combine(p_ref)
    h = jnp.where(s >= 0, s, s * a_ref[...])
    o_ref[...] = (
        jnp.dot(wt_ref[...], h, preferred_element_type=jnp.float32) + b_ref[...]
    )


def _heads_body(p_ref, wmt_ref, bm_ref, wst_ref, bs_ref, a_ref, noise_t_ref,
                zm_ref, zs_ref, z_ref, zt_ref):
    s = _combine(p_ref)
    h = jnp.where(s >= 0, s, s * a_ref[...])
    zmt = (jnp.dot(wmt_ref[...], h, preferred_element_type=jnp.float32)
           + bm_ref[...])
    v = (jnp.dot(wst_ref[...], h, preferred_element_type=jnp.float32)
         + bs_ref[...])
    zst = jnp.maximum(v, 0.0) + jnp.log(1.0 + jnp.exp(-jnp.abs(v)))
    zmt = zmt[:, :N]
    zst = zst[:, :N]
    zt = zmt + noise_t_ref[...] * zst
    zm_ref[...] = zmt.T
    zs_ref[...] = zst.T
    z_ref[...] = zt.T
    zt_ref[...] = zt


def _decode_body(z_ref, zt_ref, o_ref):
    logits = jnp.dot(z_ref[...], zt_ref[...], preferred_element_type=jnp.float32)
    o_ref[...] = 1.0 / (1.0 + jnp.exp(-logits))


_encode0 = pl.pallas_call(
    _encode0_body,
    out_shape=jax.ShapeDtypeStruct((NZ, NPAD), jnp.float32),
)

_mid = pl.pallas_call(
    _mid_body,
    out_shape=jax.ShapeDtypeStruct((NZ, NPAD), jnp.float32),
)

_heads = pl.pallas_call(
    _heads_body,
    out_shape=(
        jax.ShapeDtypeStruct((N, NZ), jnp.float32),
        jax.ShapeDtypeStruct((N, NZ), jnp.float32),
        jax.ShapeDtypeStruct((N, NZ), jnp.float32),
        jax.ShapeDtypeStruct((NZ, N), jnp.float32),
    ),
)

_decode = pl.pallas_call(
    _decode_body,
    grid=(N // BM,),
    in_specs=[
        pl.BlockSpec((BM, NZ), lambda i: (i, 0)),
        pl.BlockSpec((NZ, N), lambda i: (0, 0)),
    ],
    out_specs=pl.BlockSpec((BM, N), lambda i: (i, 0)),
    out_shape=jax.ShapeDtypeStruct((N, N), jnp.float32),
)


UNROLL = 8


def _spmm_body(ht_hbm, src_hbm, dst_hbm, adj_hbm, out_hbm,
               src_a, dst_a, adj_a, src_b, dst_b, adj_b, h_v, acc_v,
               sem_h, sem_e0, sem_e1):
    cid = lax.axis_index("c")
    sid = lax.axis_index("s")
    q = sid % NQ
    es = cid * NQ + sid // NQ
    wid = cid * NS + sid

    hcopy = pltpu.async_copy(
        ht_hbm.at[pl.ds(q * (NF * NPAD), NF * NPAD)], h_v, sem_h)

    bufs = [(src_a, dst_a, adj_a), (src_b, dst_b, adj_b)]
    sems = [sem_e0, sem_e1]

    def _start(b):
        off = es * EPW + b * BLK
        sv, dv, av = bufs[b % 2]
        sem = sems[b % 2]
        return (pltpu.async_copy(src_hbm.at[pl.ds(off, BLK)], sv, sem),
                pltpu.async_copy(dst_hbm.at[pl.ds(off, BLK)], dv, sem),
                pltpu.async_copy(adj_hbm.at[pl.ds(off, BLK)], av, sem))

    pending = {0: _start(0)}

    zero16 = jnp.zeros((16,), jnp.float32)

    def _zb(i, c):
        for u in range(8):
            acc_v[pl.ds((i * 8 + u) * 16, 16)] = zero16
        return c
    lax.fori_loop(0, (NF * NPAD) // (16 * 8), _zb, 0)
    hcopy.wait()

    lanesf = [jnp.full((16,), f * NPAD, jnp.int32) for f in range(NF)]

    for b in range(NB):
        if b + 1 < NB:
            pending[b + 1] = _start(b + 1)
        for c in pending.pop(b):
            c.wait()
        sv, dv, av = bufs[b % 2]

        @plsc.parallel_loop(0, BLK // 16, unroll=UNROLL)
        def _grp(g, sv=sv, dv=dv, av=av):
            base = g * 16
            src16 = sv[pl.ds(base, 16)]
            dst16 = dv[pl.ds(base, 16)]
            adj16 = av[pl.ds(base, 16)]
            for f in range(NF):
                vals = plsc.load_gather(h_v, [src16 + lanesf[f]]) * adj16
                plsc.addupdate_scatter(acc_v, [dst16 + lanesf[f]], vals)

    for f in range(NF):
        pltpu.sync_copy(acc_v.at[pl.ds(f * NPAD, NPAD)], out_hbm.at[wid, f])


_SPMM_CACHE = {}


def _get_spmm():
    if "k" not in _SPMM_CACHE:
        _SPMM_CACHE["k"] = pl.kernel(
            _spmm_body,
            mesh=plsc.VectorSubcoreMesh(
                core_axis_name="c", subcore_axis_name="s", num_cores=NC),
            compiler_params=pltpu.CompilerParams(needs_layout_passes=False),
            out_type=jax.ShapeDtypeStruct((NT, NF, NPAD), jnp.float32),
            scratch_types=[
                pltpu.VMEM((BLK,), jnp.int32),
                pltpu.VMEM((BLK,), jnp.int32),
                pltpu.VMEM((BLK,), jnp.float32),
                pltpu.VMEM((BLK,), jnp.int32),
                pltpu.VMEM((BLK,), jnp.int32),
                pltpu.VMEM((BLK,), jnp.float32),
                pltpu.VMEM((NF * NPAD,), jnp.float32),
                pltpu.VMEM((NF * NPAD,), jnp.float32),
                pltpu.SemaphoreType.DMA,
                pltpu.SemaphoreType.DMA,
                pltpu.SemaphoreType.DMA,
            ],
        )
    return _SPMM_CACHE["k"]


def kernel(x, edge_index, adj_vals, W0, b0, a0, W1, b1, a1, Wm, bm, Ws, bs):
    dst = edge_index[0]
    src = edge_index[1]
    pad = EP - E
    spread = jnp.arange(pad, dtype=jnp.int32) % N
    srcp = jnp.concatenate([src, spread])
    dstp = jnp.concatenate([dst, spread])
    adjp = jnp.pad(adj_vals, (0, pad))
    noise_t = jax.random.normal(jax.random.key(42), (N, NZ),
                                dtype=jnp.float32).T

    spmm = _get_spmm()

    def run_spmm(ht):
        return spmm(ht.reshape(-1), srcp, dstp, adjp)

    h0t = _encode0(x, W0, b0.reshape(1, NZ))
    parts0 = run_spmm(h0t)
    h1t = _mid(parts0, W1.T, b1.reshape(NZ, 1), a0.reshape(1, 1))
    parts1 = run_spmm(h1t)
    zm, zs, z, zt = _heads(parts1, Wm.T, bm.reshape(NZ, 1), Ws.T,
                           bs.reshape(NZ, 1), a1.reshape(1, 1), noise_t)
    a_probs = _decode(z, zt)
    return (a_probs, zm, zs)
